# baseline (speedup 1.0000x reference)
import functools

import jax
import jax.numpy as jnp
from jax import lax
from jax.experimental import pallas as pl
from jax.experimental.pallas import tpu as pltpu
from jax.experimental.pallas import tpu_sc as plsc

_B, _S = 4, 2048
_E = 1024
_NTOK = _B * _S
_NC, _NS, _L = 2, 16, 16
_NW = _NC * _NS
_PER_W = _NTOK // _NW
_CHUNK = 32
_NCHUNK = _PER_W // _CHUNK


def _sc_body(idx_hbm, table_hbm, pos_hbm, out_hbm, idx_v, buf_v, pos_v, gsem, psem):
    c = lax.axis_index("c")
    s = lax.axis_index("s")
    wid = s * _NC + c
    base = wid * _PER_W
    pos_base = lax.rem(base, _S)
    pltpu.sync_copy(idx_hbm.at[pl.ds(base, _PER_W)], idx_v)

    def chunk_step(ci, carry):
        off = ci * _CHUNK
        gather = pltpu.async_copy(
            table_hbm.at[idx_v.at[pl.ds(off, _CHUNK)]], buf_v, gsem)
        pcopy = pltpu.async_copy(
            pos_hbm.at[pl.ds(pos_base + off, _CHUNK)], pos_v, psem)
        gather.wait()
        pcopy.wait()

        def add_step(j, carry2):
            r = j // (_E // _L)
            k = lax.rem(j, _E // _L) * _L
            buf_v[r, pl.ds(k, _L)] += pos_v[r, pl.ds(k, _L)]
            return carry2

        lax.fori_loop(0, _CHUNK * (_E // _L), add_step, 0, unroll=8)
        pltpu.sync_copy(buf_v, out_hbm.at[pl.ds(base + off, _CHUNK)])
        return carry

    lax.fori_loop(0, _NCHUNK, chunk_step, 0)


@jax.jit
def _embed(idx, table, pos):
    mesh = plsc.VectorSubcoreMesh(core_axis_name="c", subcore_axis_name="s")
    fn = pl.kernel(
        _sc_body,
        out_type=jax.ShapeDtypeStruct((_NTOK, _E), jnp.float32),
        mesh=mesh,
        scratch_types=[
            pltpu.VMEM((_PER_W,), jnp.int32),
            pltpu.VMEM((_CHUNK, _E), jnp.float32),
            pltpu.VMEM((_CHUNK, _E), jnp.float32),
            pltpu.SemaphoreType.DMA,
            pltpu.SemaphoreType.DMA,
        ],
    )
    return fn(idx, table, pos)


def _position_table():
    power = jnp.arange(0, _E, 2, dtype=jnp.float32) / float(_E)
    divisor = 10000.0 ** power
    seqpos = jnp.arange(1, _S + 1, dtype=jnp.float32)
    index = seqpos[:, None] / divisor
    pos = jnp.stack((jnp.sin(index), jnp.cos(index)), axis=-1)
    return pos.reshape(_S, _E)


def kernel(inputs, table):
    idx = inputs.reshape(_NTOK).astype(jnp.int32)
    out = _embed(idx, table, _position_table())
    return out.reshape(_B, _S, _E)
